# Initial kernel scaffold; baseline (speedup 1.0000x reference)
import jax
import jax.numpy as jnp
from jax.experimental import pallas as pl


def kernel(x, W1, g1, b1, W2, g2, b2, W3, g3, b3, Wm, bm, gm, betam):
    raise NotImplementedError("write your pallas kernel here")


# speedup vs baseline: 5.3190x; 5.3190x over previous
import functools

import jax
import jax.numpy as jnp
from jax import lax
from jax.experimental import pallas as pl
from jax.experimental.pallas import tpu as pltpu
from jax.experimental.pallas import tpu_sc as plsc

KNB = 40
EPS = 1e-5
NEG = -3.0e38


def _knn_body(x_ref, xr_ref, idx_ref, *, rb):
    xb = x_ref[0]
    cdim, n = xb.shape
    xx = jnp.sum(xb * xb, axis=0, keepdims=True)
    xr = xr_ref[0]
    dot = lax.dot_general(xr, xb, (((0,), (0,)), ((), ())),
                          preferred_element_type=jnp.float32)
    q = 2.0 * dot - xx
    iota = lax.broadcasted_iota(jnp.int32, (rb, n), 1)
    for t in range(KNB):
        m = jnp.max(q, axis=1, keepdims=True)
        am = jnp.min(jnp.where(q == m, iota, n), axis=1, keepdims=True)
        idx_ref[0, :, t:t + 1] = am
        q = jnp.where(iota == am, NEG, q)


def _knn(x):
    b, c, n = x.shape
    rb = 256
    return pl.pallas_call(
        functools.partial(_knn_body, rb=rb),
        grid=(b, n // rb),
        in_specs=[pl.BlockSpec((1, c, n), lambda i, r: (i, 0, 0)),
                  pl.BlockSpec((1, c, rb), lambda i, r: (i, 0, r))],
        out_specs=pl.BlockSpec((1, rb, KNB), lambda i, r: (i, r, 0)),
        out_shape=jax.ShapeDtypeStruct((b, n, KNB), jnp.int32),
    )(x, x)


def _scdiff(x, idx):
    b, c, n = x.shape
    p = 128
    pc = n // p
    nw = 32
    u_total = b * pc
    upw = u_total // nw
    assert upw * nw == u_total
    mesh = plsc.VectorSubcoreMesh(core_axis_name="c", subcore_axis_name="s",
                                  num_cores=2, num_subcores=16)

    @functools.partial(
        pl.kernel,
        out_type=jax.ShapeDtypeStruct((b, c, KNB, n), jnp.float32),
        mesh=mesh,
        compiler_params=pltpu.CompilerParams(needs_layout_passes=False),
        scratch_types=[
            pltpu.VMEM((c, n), jnp.float32),
            pltpu.VMEM((p, KNB), jnp.int32),
            pltpu.VMEM((8, KNB, p), jnp.float32),
        ],
    )
    def gk(x_h, idx_h, fd_h, table_v, idx_v, fd_v):
        wid = lax.axis_index("s") * 2 + lax.axis_index("c")
        for ui in range(upw):
            u = wid * upw + ui
            pci = lax.rem(u, pc)
            bi = u // pc
            n0 = pci * p
            if ui == 0:
                pltpu.sync_copy(x_h.at[bi], table_v)
            pltpu.sync_copy(idx_h.at[bi, pl.ds(n0, p), :], idx_v)

            def ccbody(cc8, carry):
                for pg in range(p // 16):
                    pvec = lax.iota(jnp.int32, 16) + pg * 16

                    def clbody(cl, carry2, pvec=pvec, pg=pg, cc8=cc8):
                        ci = cc8 * 8 + cl
                        cvec = jnp.full((16,), ci, jnp.int32)
                        xn = table_v[ci, pl.ds(n0 + pg * 16, 16)]
                        for k in range(KNB):
                            iv = plsc.load_gather(
                                idx_v, [pvec, jnp.full((16,), k, jnp.int32)])
                            vj = plsc.load_gather(table_v, [cvec, iv])
                            fd_v[cl, k, pl.ds(pg * 16, 16)] = vj - xn
                        return carry2

                    lax.fori_loop(0, 8, clbody, 0)
                pltpu.sync_copy(
                    fd_v, fd_h.at[bi, pl.ds(cc8 * 8, 8), :, pl.ds(n0, p)])
                return carry

            lax.fori_loop(0, c // 8, ccbody, 0)

    return gk(x, idx)


def _edge_body(fd_ref, x_ref, w_ref, mx_ref, mn_ref, sm_ref, sq_ref):
    _, c, k, p = fd_ref.shape
    co = w_ref.shape[0]
    fdm = fd_ref[0].reshape(c, k * p)
    xnb = jnp.broadcast_to(x_ref[0][:, None, :], (c, k, p)).reshape(c, k * p)
    f = jnp.concatenate([fdm, xnb], axis=0)
    h = lax.dot_general(w_ref[...], f, (((1,), (0,)), ((), ())),
                        preferred_element_type=jnp.float32)
    h3 = h.reshape(co, k, p)
    mx_ref[0] = jnp.max(h3, axis=1)
    mn_ref[0] = jnp.min(h3, axis=1)
    sm_ref[0] = jnp.sum(h3, axis=1)
    sq_ref[0] = jnp.sum(h3 * h3, axis=1)


def _edge(fd, x, w):
    b, c, k, n = fd.shape
    co = w.shape[0]
    p = 128
    os = pl.BlockSpec((1, co, p), lambda i, r: (i, 0, r))
    osh = jax.ShapeDtypeStruct((b, co, n), jnp.float32)
    return pl.pallas_call(
        _edge_body,
        grid=(b, n // p),
        in_specs=[
            pl.BlockSpec((1, c, k, p), lambda i, r: (i, 0, 0, r)),
            pl.BlockSpec((1, c, p), lambda i, r: (i, 0, r)),
            pl.BlockSpec((co, 2 * c), lambda i, r: (0, 0)),
        ],
        out_specs=[os, os, os, os],
        out_shape=[osh, osh, osh, osh],
    )(fd, x, w)


def _post_body(mx_ref, mn_ref, sm_ref, sq_ref, g_ref, b_ref, out_ref):
    _, c, n = mx_ref.shape
    cg = c // 2
    cnt = cg * n * float(KNB)
    stats = []
    for gi in range(2):
        sl = slice(gi * cg, (gi + 1) * cg)
        s1 = jnp.sum(sm_ref[0, sl, :])
        s2 = jnp.sum(sq_ref[0, sl, :])
        mean = s1 / cnt
        var = s2 / cnt - mean * mean
        stats.append((mean, jnp.sqrt(var + EPS)))
    ci = lax.broadcasted_iota(jnp.int32, (c, 1), 0)
    mean_c = jnp.where(ci < cg, stats[0][0], stats[1][0])
    std_c = jnp.where(ci < cg, stats[0][1], stats[1][1])
    gcol = g_ref[...]
    nmax = (mx_ref[0] - mean_c) / std_c
    nmin = (mn_ref[0] - mean_c) / std_c
    y = gcol * jnp.where(gcol >= 0, nmax, nmin) + b_ref[...]
    out_ref[0] = jnp.where(y >= 0, y, 0.2 * y)


def _post(mx, mn, sm, sq, g, bta):
    b, c, n = mx.shape
    fs = pl.BlockSpec((1, c, n), lambda i: (i, 0, 0))
    cs = pl.BlockSpec((c, 1), lambda i: (0, 0))
    return pl.pallas_call(
        _post_body,
        grid=(b,),
        in_specs=[fs, fs, fs, fs, cs, cs],
        out_specs=fs,
        out_shape=jax.ShapeDtypeStruct((b, c, n), jnp.float32),
    )(mx, mn, sm, sq, g, bta)


def _final_body(xf_ref, wm_ref, bm_ref, gm_ref, bt_ref, out_ref):
    co = wm_ref.shape[0]
    n = xf_ref.shape[2]
    h = lax.dot_general(wm_ref[...], xf_ref[0], (((1,), (0,)), ((), ())),
                        preferred_element_type=jnp.float32)
    h = h + bm_ref[...]
    hmax = jnp.max(h, axis=1, keepdims=True)
    hmin = jnp.min(h, axis=1, keepdims=True)
    hsum = jnp.sum(h, axis=1, keepdims=True)
    hsq = jnp.sum(h * h, axis=1, keepdims=True)
    cg = co // 8
    cnt = float(cg * n)
    means, stds = [], []
    for gi in range(8):
        sl = slice(gi * cg, (gi + 1) * cg)
        mean = jnp.sum(hsum[sl, :]) / cnt
        var = jnp.sum(hsq[sl, :]) / cnt - mean * mean
        means.append(mean)
        stds.append(jnp.sqrt(var + EPS))
    ci = lax.broadcasted_iota(jnp.int32, (co, 1), 0) // cg
    mean_c = jnp.zeros((co, 1), jnp.float32)
    std_c = jnp.zeros((co, 1), jnp.float32)
    for gi in range(8):
        mean_c = jnp.where(ci == gi, means[gi], mean_c)
        std_c = jnp.where(ci == gi, stds[gi], std_c)
    gcol = gm_ref[...]
    nmax = (hmax - mean_c) / std_c
    nmin = (hmin - mean_c) / std_c
    y = gcol * jnp.where(gcol >= 0, nmax, nmin) + bt_ref[...]
    out_ref[0] = jnp.maximum(y, 0.0)


def _final(xf, wm, bm, gm, bt):
    b, cin, n = xf.shape
    co = wm.shape[0]
    cs = pl.BlockSpec((co, 1), lambda i: (0, 0))
    out = pl.pallas_call(
        _final_body,
        grid=(b,),
        in_specs=[
            pl.BlockSpec((1, cin, n), lambda i: (i, 0, 0)),
            pl.BlockSpec((co, cin), lambda i: (0, 0)),
            cs, cs, cs,
        ],
        out_specs=pl.BlockSpec((1, co, 1), lambda i: (i, 0, 0)),
        out_shape=jax.ShapeDtypeStruct((b, co, 1), jnp.float32),
    )(xf, wm, bm, gm, bt)
    return out.reshape(b, co)


def _layer(x, idx, w, g, bta):
    fd = _scdiff(x, idx)
    mx, mn, sm, sq = _edge(fd, x, w)
    return _post(mx, mn, sm, sq, g.reshape(-1, 1), bta.reshape(-1, 1))


def kernel(x, W1, g1, b1, W2, g2, b2, W3, g3, b3, Wm, bm, gm, betam):
    b, c0, n = x.shape
    x0 = jnp.pad(x, ((0, 0), (0, 8 - c0), (0, 0)))
    w1 = jnp.concatenate(
        [jnp.pad(W1[:, :c0], ((0, 0), (0, 8 - c0))),
         jnp.pad(W1[:, c0:], ((0, 0), (0, 8 - c0)))], axis=1)
    idx1 = _knn(x0)
    x1 = _layer(x0, idx1, w1, g1, b1)
    idx2 = _knn(x1)
    x2 = _layer(x1, idx2, W2, g2, b2)
    x3 = _layer(x2, idx2, W3, g3, b3)
    xf = jnp.concatenate([x1, x2, x3], axis=1)
    x4 = _final(xf, Wm, bm.reshape(-1, 1), gm.reshape(-1, 1),
                betam.reshape(-1, 1))
    return x4, xf


# speedup vs baseline: 7.8289x; 1.4719x over previous
import functools

import jax
import jax.numpy as jnp
from jax import lax
from jax.experimental import pallas as pl
from jax.experimental.pallas import tpu as pltpu
from jax.experimental.pallas import tpu_sc as plsc

KNB = 40
EPS = 1e-5
NEG = -3.0e38


def _knn_body(x_ref, xr_ref, idx_ref, *, rb):
    xb = x_ref[0]
    cdim, n = xb.shape
    xx = jnp.sum(xb * xb, axis=0, keepdims=True)
    xr = xr_ref[0]
    dot = lax.dot_general(xr, xb, (((0,), (0,)), ((), ())),
                          preferred_element_type=jnp.float32)
    q = 2.0 * dot - xx
    iota = lax.broadcasted_iota(jnp.int32, (rb, n), 1)
    for t in range(KNB):
        m = jnp.max(q, axis=1, keepdims=True)
        am = jnp.min(jnp.where(q == m, iota, n), axis=1, keepdims=True)
        idx_ref[0, :, t:t + 1] = am
        q = jnp.where(iota == am, NEG, q)


def _knn(x):
    b, c, n = x.shape
    rb = 256
    return pl.pallas_call(
        functools.partial(_knn_body, rb=rb),
        grid=(b, n // rb),
        in_specs=[pl.BlockSpec((1, c, n), lambda i, r: (i, 0, 0)),
                  pl.BlockSpec((1, c, rb), lambda i, r: (i, 0, r))],
        out_specs=pl.BlockSpec((1, rb, KNB), lambda i, r: (i, r, 0)),
        out_shape=jax.ShapeDtypeStruct((b, n, KNB), jnp.int32),
    )(x, x)


def _scdiff(x, idx):
    b, c, n = x.shape
    p = 128
    pc = n // p
    nw = 32
    u_total = b * pc
    upw = u_total // nw
    assert upw * nw == u_total
    mesh = plsc.VectorSubcoreMesh(core_axis_name="c", subcore_axis_name="s",
                                  num_cores=2, num_subcores=16)

    @functools.partial(
        pl.kernel,
        out_type=jax.ShapeDtypeStruct((b, c, KNB, n), jnp.float32),
        mesh=mesh,
        compiler_params=pltpu.CompilerParams(needs_layout_passes=False),
        scratch_types=[
            pltpu.VMEM((c, n), jnp.float32),
            pltpu.VMEM((p, KNB), jnp.int32),
            pltpu.VMEM((8, KNB, p), jnp.float32),
        ],
    )
    def gk(x_h, idx_h, fd_h, table_v, idx_v, fd_v):
        wid = lax.axis_index("s") * 2 + lax.axis_index("c")
        for ui in range(upw):
            u = wid * upw + ui
            pci = lax.rem(u, pc)
            bi = u // pc
            n0 = pci * p
            if ui == 0:
                pltpu.sync_copy(x_h.at[bi], table_v)
            pltpu.sync_copy(idx_h.at[bi, pl.ds(n0, p), :], idx_v)

            def ccbody(cc8, carry):
                def pgbody(pg, carry2, cc8=cc8):
                    pvec = lax.iota(jnp.int32, 16) + pg * 16
                    cvs = [jnp.full((16,), cc8 * 8 + i, jnp.int32)
                           for i in range(8)]
                    xns = [table_v[cc8 * 8 + i,
                                   pl.ds(n0 + pg * 16, 16)]
                           for i in range(8)]
                    for k in range(KNB):
                        iv = plsc.load_gather(
                            idx_v, [pvec, jnp.full((16,), k, jnp.int32)])
                        for i in range(8):
                            vj = plsc.load_gather(table_v, [cvs[i], iv])
                            fd_v[i, k, pl.ds(pg * 16, 16)] = vj - xns[i]
                    return carry2

                lax.fori_loop(0, p // 16, pgbody, 0)
                pltpu.sync_copy(
                    fd_v, fd_h.at[bi, pl.ds(cc8 * 8, 8), :, pl.ds(n0, p)])
                return carry

            lax.fori_loop(0, c // 8, ccbody, 0)

    return gk(x, idx)


def _edge_body(fd_ref, x_ref, w_ref, mx_ref, mn_ref, sm_ref, sq_ref):
    _, c, k, p = fd_ref.shape
    co = w_ref.shape[0]
    fdm = fd_ref[0].reshape(c, k * p)
    xnb = jnp.broadcast_to(x_ref[0][:, None, :], (c, k, p)).reshape(c, k * p)
    f = jnp.concatenate([fdm, xnb], axis=0)
    h = lax.dot_general(w_ref[...], f, (((1,), (0,)), ((), ())),
                        preferred_element_type=jnp.float32)
    h3 = h.reshape(co, k, p)
    mx_ref[0] = jnp.max(h3, axis=1)
    mn_ref[0] = jnp.min(h3, axis=1)
    sm_ref[0] = jnp.sum(h3, axis=1)
    sq_ref[0] = jnp.sum(h3 * h3, axis=1)


def _edge(fd, x, w):
    b, c, k, n = fd.shape
    co = w.shape[0]
    p = 128
    os = pl.BlockSpec((1, co, p), lambda i, r: (i, 0, r))
    osh = jax.ShapeDtypeStruct((b, co, n), jnp.float32)
    return pl.pallas_call(
        _edge_body,
        grid=(b, n // p),
        in_specs=[
            pl.BlockSpec((1, c, k, p), lambda i, r: (i, 0, 0, r)),
            pl.BlockSpec((1, c, p), lambda i, r: (i, 0, r)),
            pl.BlockSpec((co, 2 * c), lambda i, r: (0, 0)),
        ],
        out_specs=[os, os, os, os],
        out_shape=[osh, osh, osh, osh],
    )(fd, x, w)


def _post_body(mx_ref, mn_ref, sm_ref, sq_ref, g_ref, b_ref, out_ref):
    _, c, n = mx_ref.shape
    cg = c // 2
    cnt = cg * n * float(KNB)
    stats = []
    for gi in range(2):
        sl = slice(gi * cg, (gi + 1) * cg)
        s1 = jnp.sum(sm_ref[0, sl, :])
        s2 = jnp.sum(sq_ref[0, sl, :])
        mean = s1 / cnt
        var = s2 / cnt - mean * mean
        stats.append((mean, jnp.sqrt(var + EPS)))
    ci = lax.broadcasted_iota(jnp.int32, (c, 1), 0)
    mean_c = jnp.where(ci < cg, stats[0][0], stats[1][0])
    std_c = jnp.where(ci < cg, stats[0][1], stats[1][1])
    gcol = g_ref[...]
    nmax = (mx_ref[0] - mean_c) / std_c
    nmin = (mn_ref[0] - mean_c) / std_c
    y = gcol * jnp.where(gcol >= 0, nmax, nmin) + b_ref[...]
    out_ref[0] = jnp.where(y >= 0, y, 0.2 * y)


def _post(mx, mn, sm, sq, g, bta):
    b, c, n = mx.shape
    fs = pl.BlockSpec((1, c, n), lambda i: (i, 0, 0))
    cs = pl.BlockSpec((c, 1), lambda i: (0, 0))
    return pl.pallas_call(
        _post_body,
        grid=(b,),
        in_specs=[fs, fs, fs, fs, cs, cs],
        out_specs=fs,
        out_shape=jax.ShapeDtypeStruct((b, c, n), jnp.float32),
    )(mx, mn, sm, sq, g, bta)


def _final_body(xf_ref, wm_ref, bm_ref, gm_ref, bt_ref, out_ref):
    co = wm_ref.shape[0]
    n = xf_ref.shape[2]
    h = lax.dot_general(wm_ref[...], xf_ref[0], (((1,), (0,)), ((), ())),
                        preferred_element_type=jnp.float32)
    h = h + bm_ref[...]
    hmax = jnp.max(h, axis=1, keepdims=True)
    hmin = jnp.min(h, axis=1, keepdims=True)
    hsum = jnp.sum(h, axis=1, keepdims=True)
    hsq = jnp.sum(h * h, axis=1, keepdims=True)
    cg = co // 8
    cnt = float(cg * n)
    means, stds = [], []
    for gi in range(8):
        sl = slice(gi * cg, (gi + 1) * cg)
        mean = jnp.sum(hsum[sl, :]) / cnt
        var = jnp.sum(hsq[sl, :]) / cnt - mean * mean
        means.append(mean)
        stds.append(jnp.sqrt(var + EPS))
    ci = lax.broadcasted_iota(jnp.int32, (co, 1), 0) // cg
    mean_c = jnp.zeros((co, 1), jnp.float32)
    std_c = jnp.zeros((co, 1), jnp.float32)
    for gi in range(8):
        mean_c = jnp.where(ci == gi, means[gi], mean_c)
        std_c = jnp.where(ci == gi, stds[gi], std_c)
    gcol = gm_ref[...]
    nmax = (hmax - mean_c) / std_c
    nmin = (hmin - mean_c) / std_c
    y = gcol * jnp.where(gcol >= 0, nmax, nmin) + bt_ref[...]
    out_ref[0] = jnp.maximum(y, 0.0)


def _final(xf, wm, bm, gm, bt):
    b, cin, n = xf.shape
    co = wm.shape[0]
    cs = pl.BlockSpec((co, 1), lambda i: (0, 0))
    out = pl.pallas_call(
        _final_body,
        grid=(b,),
        in_specs=[
            pl.BlockSpec((1, cin, n), lambda i: (i, 0, 0)),
            pl.BlockSpec((co, cin), lambda i: (0, 0)),
            cs, cs, cs,
        ],
        out_specs=pl.BlockSpec((1, co, 1), lambda i: (i, 0, 0)),
        out_shape=jax.ShapeDtypeStruct((b, co, 1), jnp.float32),
    )(xf, wm, bm, gm, bt)
    return out.reshape(b, co)


def _layer(x, idx, w, g, bta):
    fd = _scdiff(x, idx)
    mx, mn, sm, sq = _edge(fd, x, w)
    return _post(mx, mn, sm, sq, g.reshape(-1, 1), bta.reshape(-1, 1))


def kernel(x, W1, g1, b1, W2, g2, b2, W3, g3, b3, Wm, bm, gm, betam):
    b, c0, n = x.shape
    x0 = jnp.pad(x, ((0, 0), (0, 8 - c0), (0, 0)))
    w1 = jnp.concatenate(
        [jnp.pad(W1[:, :c0], ((0, 0), (0, 8 - c0))),
         jnp.pad(W1[:, c0:], ((0, 0), (0, 8 - c0)))], axis=1)
    idx1 = _knn(x0)
    x1 = _layer(x0, idx1, w1, g1, b1)
    idx2 = _knn(x1)
    x2 = _layer(x1, idx2, W2, g2, b2)
    x3 = _layer(x2, idx2, W3, g3, b3)
    xf = jnp.concatenate([x1, x2, x3], axis=1)
    x4 = _final(xf, Wm, bm.reshape(-1, 1), gm.reshape(-1, 1),
                betam.reshape(-1, 1))
    return x4, xf
